# Initial kernel scaffold; baseline (speedup 1.0000x reference)
import jax
import jax.numpy as jnp
from jax.experimental import pallas as pl


def kernel(x, emb_table):
    raise NotImplementedError("write your pallas kernel here")


# speedup vs baseline: 8.1372x; 8.1372x over previous
import functools

import jax
import jax.numpy as jnp
from jax import lax
from jax.experimental import pallas as pl
from jax.experimental.pallas import tpu as pltpu
from jax.experimental.pallas import tpu_sc as plsc


def kernel(x, emb_table):
    B0, S = x.shape
    V, D = emb_table.shape
    B = B0 * S

    info = plsc.get_sparse_core_info()
    NC, NS = info.num_cores, info.num_subcores
    NW = NC * NS

    K = 128
    NK = 4
    CH = K * NK
    b_per_w = B // NW
    n_chunks = b_per_w // CH

    idx2d = x.reshape(B // K, K).astype(jnp.int32)

    mesh = plsc.VectorSubcoreMesh(core_axis_name="c", subcore_axis_name="s")

    @functools.partial(
        pl.kernel,
        mesh=mesh,
        out_type=jax.ShapeDtypeStruct((B, D), jnp.float32),
        scratch_types=[
            pltpu.VMEM((NK, K), jnp.int32),
            pltpu.VMEM((CH, D), jnp.float32),
            pltpu.SemaphoreType.DMA,
        ],
    )
    def gather_kernel(table_hbm, idx_hbm, out_hbm, idx_v, rows_v, sem):
        wid = lax.axis_index("s") * NC + lax.axis_index("c")
        idx_row0 = wid * (b_per_w // K)
        out_base = wid * b_per_w

        def body(i, carry):
            pltpu.sync_copy(idx_hbm.at[pl.ds(idx_row0 + i * NK, NK)], idx_v)
            for j in range(NK):
                pltpu.async_copy(
                    table_hbm.at[idx_v.at[j]],
                    rows_v.at[pl.ds(j * K, K)],
                    sem,
                )
            for j in range(NK):
                pltpu.make_async_copy(
                    table_hbm.at[idx_v.at[j]],
                    rows_v.at[pl.ds(j * K, K)],
                    sem,
                ).wait()
            pltpu.sync_copy(rows_v, out_hbm.at[pl.ds(out_base + i * CH, CH)])
            return carry

        lax.fori_loop(0, n_chunks, body, 0)

    out = gather_kernel(emb_table, idx2d)
    return out.reshape(B0, S, D)


# speedup vs baseline: 8.2607x; 1.0152x over previous
import functools

import jax
import jax.numpy as jnp
from jax import lax
from jax.experimental import pallas as pl
from jax.experimental.pallas import tpu as pltpu
from jax.experimental.pallas import tpu_sc as plsc


def kernel(x, emb_table):
    B0, S = x.shape
    V, D = emb_table.shape
    B = B0 * S

    info = plsc.get_sparse_core_info()
    NC, NS = info.num_cores, info.num_subcores
    NW = NC * NS

    K = 128
    NK = 2
    CH = K * NK
    b_per_w = B // NW
    n_chunks = b_per_w // CH

    idx2d = x.reshape(B // K, K).astype(jnp.int32)

    mesh = plsc.VectorSubcoreMesh(core_axis_name="c", subcore_axis_name="s")

    @functools.partial(
        pl.kernel,
        mesh=mesh,
        out_type=jax.ShapeDtypeStruct((B, D), jnp.float32),
        scratch_types=[
            pltpu.VMEM((NK, K), jnp.int32),
            pltpu.VMEM((NK, K), jnp.int32),
            pltpu.VMEM((CH, D), jnp.float32),
            pltpu.VMEM((CH, D), jnp.float32),
            pltpu.SemaphoreType.DMA,
            pltpu.SemaphoreType.DMA,
            pltpu.SemaphoreType.DMA,
        ],
    )
    def gather_kernel(table_hbm, idx_hbm, out_hbm,
                      idx_v0, idx_v1, rows_v0, rows_v1, gsem, ssem0, ssem1):
        wid = lax.axis_index("s") * NC + lax.axis_index("c")
        idx_row0 = wid * (b_per_w // K)
        out_base = wid * b_per_w

        bufs = ((idx_v0, rows_v0, ssem0), (idx_v1, rows_v1, ssem1))

        def stage_and_fire(i, b):
            idx_v, rows_v, _ = bufs[b]
            pltpu.sync_copy(idx_hbm.at[pl.ds(idx_row0 + i * NK, NK)], idx_v)
            for j in range(NK):
                pltpu.async_copy(table_hbm.at[idx_v.at[j]],
                                 rows_v.at[pl.ds(j * K, K)], gsem)

        def wait_gather(b):
            idx_v, rows_v, _ = bufs[b]
            for j in range(NK):
                pltpu.make_async_copy(table_hbm.at[idx_v.at[j]],
                                      rows_v.at[pl.ds(j * K, K)], gsem).wait()

        def fire_scatter(i, b):
            _, rows_v, ssem = bufs[b]
            pltpu.async_copy(rows_v, out_hbm.at[pl.ds(out_base + i * CH, CH)],
                             ssem)

        def wait_scatter(b):
            _, rows_v, ssem = bufs[b]
            pltpu.make_async_copy(rows_v, out_hbm.at[pl.ds(out_base, CH)],
                                  ssem).wait()

        stage_and_fire(0, 0)
        wait_gather(0)
        fire_scatter(0, 0)
        stage_and_fire(1, 1)

        def body(t, carry):
            for b, di in ((1, 0), (0, 1)):
                i = 1 + 2 * t + di
                wait_gather(b)
                fire_scatter(i, b)
                wait_scatter(1 - b)
                stage_and_fire(i + 1, 1 - b)
            return carry

        lax.fori_loop(0, (n_chunks - 2) // 2, body, 0)

        wait_gather(1)
        fire_scatter(n_chunks - 1, 1)
        wait_scatter(0)
        wait_scatter(1)

    out = gather_kernel(emb_table, idx2d)
    return out.reshape(B0, S, D)


# speedup vs baseline: 9.1547x; 1.1082x over previous
import functools

import jax
import jax.numpy as jnp
from jax import lax
from jax.experimental import pallas as pl
from jax.experimental.pallas import tpu as pltpu
from jax.experimental.pallas import tpu_sc as plsc


def kernel(x, emb_table):
    B0, S = x.shape
    V, D = emb_table.shape
    B = B0 * S

    info = plsc.get_sparse_core_info()
    NC, NS = info.num_cores, info.num_subcores
    NW = NC * NS

    K = 128
    R = 4
    b_per_w = B // NW
    n_chunks = b_per_w // K

    idx2d = x.reshape(B // K, K).astype(jnp.int32)

    mesh = plsc.VectorSubcoreMesh(core_axis_name="c", subcore_axis_name="s")

    @functools.partial(
        pl.kernel,
        mesh=mesh,
        out_type=jax.ShapeDtypeStruct((B, D), jnp.float32),
        scratch_types=(
            [pltpu.VMEM((n_chunks, K), jnp.int32)]
            + [pltpu.VMEM((K, D), jnp.float32) for _ in range(R)]
            + [pltpu.SemaphoreType.DMA for _ in range(2 * R)]
        ),
    )
    def gather_kernel(table_hbm, idx_hbm, out_hbm, idx_all, *bufs_and_sems):
        rows = bufs_and_sems[:R]
        gsem = bufs_and_sems[R:2 * R]
        ssem = bufs_and_sems[2 * R:]

        wid = lax.axis_index("s") * NC + lax.axis_index("c")
        idx_row0 = wid * n_chunks
        out_base = wid * b_per_w

        pltpu.sync_copy(idx_hbm.at[pl.ds(idx_row0, n_chunks)], idx_all)

        def fire_gather(i, r):
            pltpu.async_copy(table_hbm.at[idx_all.at[i]], rows[r], gsem[r])

        def wait_gather(r):
            pltpu.make_async_copy(table_hbm.at[idx_all.at[0]], rows[r],
                                  gsem[r]).wait()

        def fire_scatter(i, r):
            pltpu.async_copy(rows[r], out_hbm.at[pl.ds(out_base + i * K, K)],
                             ssem[r])

        def wait_scatter(r):
            pltpu.make_async_copy(rows[r], out_hbm.at[pl.ds(out_base, K)],
                                  ssem[r]).wait()

        for i in range(R - 1):
            fire_gather(i, i)
        for r in range(R):
            wait_gather(r)
            fire_scatter(r, r)
            rp = (r + R - 1) % R
            if r > 0:
                wait_scatter(rp)
            fire_gather(r + R - 1, rp)

        def body(t, carry):
            for r in range(R):
                i = R * t + r
                wait_gather(r)
                fire_scatter(i, r)
                rp = (r + R - 1) % R
                wait_scatter(rp)
                fire_gather(i + R - 1, rp)
            return carry

        lax.fori_loop(1, n_chunks // R - 1, body, 0)

        base = n_chunks - R
        for r in range(R):
            wait_gather(r)
            fire_scatter(base + r, r)
            if r == 0:
                rp = R - 1
                wait_scatter(rp)
                fire_gather(n_chunks - 1, rp)
        for r in range(R):
            wait_scatter(r)

    out = gather_kernel(emb_table, idx2d)
    return out.reshape(B0, S, D)
